# baseline (speedup 1.0000x reference)
import jax
import jax.numpy as jnp
from jax import lax
from jax.experimental import pallas as pl
from jax.experimental.pallas import tpu as pltpu
from jax.experimental.pallas import tpu_sc as plsc

N_BITS = 20
BATCH = 16384
TABLE_SIZE = 1 << N_BITS
FLAT = N_BITS * TABLE_SIZE

NC = 2
NS = 16

CW = 2048
NCK = TABLE_SIZE // CW
CK_S = NCK // NS

TOKB = BATCH // NS
NROW0 = 12
NROW1 = 8
CHUNK = 128
GROUP = 16
CPR = TOKB // CHUNK


def _relayout_pass(table_ref, staging_ref, buf0, buf1, rsem, wsem,
                   sid, row0, nrows, prows):
    ck0 = sid * CK_S
    bufs = (buf0, buf1)

    def read(k, buf):
        col = pl.multiple_of((ck0 + lax.rem(k, CK_S)) * CW, CW)
        return pltpu.async_copy(
            table_ref.at[pl.ds(row0, nrows), pl.ds(col, CW)],
            buf.at[pl.ds(0, nrows)], rsem)

    def read_wait(k, buf):
        col = pl.multiple_of((ck0 + lax.rem(k, CK_S)) * CW, CW)
        pltpu.make_async_copy(
            table_ref.at[pl.ds(row0, nrows), pl.ds(col, CW)],
            buf.at[pl.ds(0, nrows)], rsem).wait()

    read(0, buf0)
    read(1, buf1)

    def step(k2, carry):
        for b in range(2):
            k = k2 * 2 + b
            cur = bufs[b]
            col = pl.multiple_of((ck0 + k) * CW, CW)
            read_wait(k, cur)
            writes = []
            for j, p in enumerate(prows):
                writes.append(pltpu.async_copy(
                    cur.at[j],
                    staging_ref.at[pl.ds(p * TABLE_SIZE + col, CW)],
                    wsem))
            for w in writes:
                w.wait()
            read(k + 2, cur)
        return carry

    lax.fori_loop(0, CK_S // 2, step, 0)
    read_wait(0, buf0)
    read_wait(1, buf1)


def _sc_body(table_ref, bitst_ref, out_ref, staging_ref,
             buf0, buf1, bits_v, idx_v, vals_v, rsem, wsem, sem):
    cc = lax.axis_index("c")
    sid = lax.axis_index("s")

    @pl.when(cc == 0)
    def _():
        _relayout_pass(table_ref, staging_ref, buf0, buf1, rsem, wsem,
                       sid, 0, 8, list(range(0, 8)))
        _relayout_pass(table_ref, staging_ref, buf0, buf1, rsem, wsem,
                       sid, 16, 4, list(range(16, 20)))

    @pl.when(cc == 1)
    def _():
        _relayout_pass(table_ref, staging_ref, buf0, buf1, rsem, wsem,
                       sid, 8, 8, list(range(8, 16)))

    plsc.subcore_barrier()

    tok0 = sid * TOKB
    pltpu.sync_copy(bitst_ref.at[:, pl.ds(tok0, TOKB)], bits_v)

    def compute_group(g, carry):
        o = pl.multiple_of(g * 16, 16)
        addr = jnp.zeros((16,), jnp.int32)
        for i in range(N_BITS):
            addr = addr + bits_v[i, pl.ds(o, 16)] * (1 << i)
        for j in range(NROW0):
            p0 = j if j < 8 else 16 + (j - 8)
            p1 = 8 + j if j < 8 else 0
            p = jnp.where(cc == 0, p0, p1)
            idx_v[j, pl.ds(o, 16)] = addr + (p << N_BITS)
        return carry

    lax.fori_loop(0, TOKB // 16, compute_group, 0)

    def gather_group(t, carry):
        copies = []
        for u in range(GROUP):
            k = t * GROUP + u
            j = lax.div(k, CPR)
            o = pl.multiple_of(lax.rem(k, CPR) * CHUNK, CHUNK)
            copies.append(pltpu.async_copy(
                staging_ref.at[idx_v.at[j, pl.ds(o, CHUNK)]],
                vals_v.at[j, pl.ds(o, CHUNK)], sem))
        for cp in copies:
            cp.wait()
        return carry

    lax.fori_loop(0, 8 * CPR // GROUP, gather_group, 0)

    @pl.when(cc == 0)
    def _():
        lax.fori_loop(8 * CPR // GROUP, NROW0 * CPR // GROUP,
                      gather_group, 0)
        pltpu.sync_copy(vals_v.at[pl.ds(0, 8)],
                        out_ref.at[pl.ds(0, 8), pl.ds(tok0, TOKB)])
        pltpu.sync_copy(vals_v.at[pl.ds(8, 4)],
                        out_ref.at[pl.ds(16, 4), pl.ds(tok0, TOKB)])

    @pl.when(cc == 1)
    def _():
        pltpu.sync_copy(vals_v.at[pl.ds(0, 8)],
                        out_ref.at[pl.ds(8, 8), pl.ds(tok0, TOKB)])


def kernel(bits, tables):
    mesh = plsc.VectorSubcoreMesh(core_axis_name="c", subcore_axis_name="s")
    run = pl.kernel(
        _sc_body,
        mesh=mesh,
        out_type=(
            jax.ShapeDtypeStruct((N_BITS, BATCH), jnp.float32),
            jax.ShapeDtypeStruct((FLAT,), jnp.float32),
        ),
        scratch_types=[
            pltpu.VMEM((8, CW), jnp.float32),
            pltpu.VMEM((8, CW), jnp.float32),
            pltpu.VMEM((N_BITS, TOKB), jnp.int32),
            pltpu.VMEM((NROW0, TOKB), jnp.int32),
            pltpu.VMEM((NROW0, TOKB), jnp.float32),
            pltpu.SemaphoreType.DMA,
            pltpu.SemaphoreType.DMA,
            pltpu.SemaphoreType.DMA,
        ],
    )
    out_t, _ = run(tables, bits.T)
    return out_t.T
